# Initial kernel scaffold; baseline (speedup 1.0000x reference)
import jax
import jax.numpy as jnp
from jax.experimental import pallas as pl


def kernel(x, k, embed):
    raise NotImplementedError("write your pallas kernel here")


# speedup vs baseline: 47.4322x; 47.4322x over previous
import functools

import jax
import jax.numpy as jnp
from jax import lax
from jax.experimental import pallas as pl
from jax.experimental.pallas import tpu as pltpu
from jax.experimental.pallas import tpu_sc as plsc

_NT = 256
_KT = 2048
_BIG = 2 ** 30


def _dist_body(x_ref, e_ref, dist_ref, i1_ref, i2_ref, b1, bi1, b2, bi2):
    j = pl.program_id(1)
    nk = pl.num_programs(1)
    x = x_ref[...]
    e = e_ref[...]
    x2 = jnp.sum(x * x, axis=1, keepdims=True)
    y2 = jnp.sum(e * e, axis=1)[None, :]
    xy = lax.dot_general(x, e, (((1,), (1,)), ((), ())),
                         preferred_element_type=jnp.float32)
    sq = (x2 + y2) + (xy * -2.0)
    dist = -jnp.sqrt(jnp.maximum(sq, 0.0))
    dist_ref[...] = dist

    iota = lax.broadcasted_iota(jnp.int32, dist.shape, 1) + j * _KT
    v1 = jnp.max(dist, axis=1, keepdims=True)
    i1 = jnp.min(jnp.where(dist == v1, iota, _BIG), axis=1, keepdims=True)
    masked = jnp.where(iota == i1, -jnp.inf, dist)
    v2 = jnp.max(masked, axis=1, keepdims=True)
    i2 = jnp.min(jnp.where(masked == v2, iota, _BIG), axis=1, keepdims=True)

    @pl.when(j == 0)
    def _():
        b1[...] = v1
        bi1[...] = i1
        b2[...] = v2
        bi2[...] = i2

    @pl.when(j > 0)
    def _():
        rb1, ri1, rb2, ri2 = b1[...], bi1[...], b2[...], bi2[...]
        first_run = rb1 >= v1
        nb1 = jnp.where(first_run, rb1, v1)
        ni1 = jnp.where(first_run, ri1, i1)
        s_a = rb2 >= v1
        sva = jnp.where(s_a, rb2, v1)
        sia = jnp.where(s_a, ri2, i1)
        s_b = rb1 >= v2
        svb = jnp.where(s_b, rb1, v2)
        sib = jnp.where(s_b, ri1, i2)
        b1[...] = nb1
        bi1[...] = ni1
        b2[...] = jnp.where(first_run, sva, svb)
        bi2[...] = jnp.where(first_run, sia, sib)

    @pl.when(j == nk - 1)
    def _():
        i1_ref[...] = bi1[...]
        i2_ref[...] = bi2[...]


def _dist_topk(flat, emb):
    n, d = flat.shape
    kc = emb.shape[0]
    return pl.pallas_call(
        _dist_body,
        grid=(n // _NT, kc // _KT),
        in_specs=[
            pl.BlockSpec((_NT, d), lambda i, j: (i, 0)),
            pl.BlockSpec((_KT, d), lambda i, j: (j, 0)),
        ],
        out_specs=[
            pl.BlockSpec((_NT, _KT), lambda i, j: (i, j)),
            pl.BlockSpec((_NT, 1), lambda i, j: (i, 0)),
            pl.BlockSpec((_NT, 1), lambda i, j: (i, 0)),
        ],
        out_shape=[
            jax.ShapeDtypeStruct((n, kc), jnp.float32),
            jax.ShapeDtypeStruct((n, 1), jnp.int32),
            jax.ShapeDtypeStruct((n, 1), jnp.int32),
        ],
        scratch_shapes=[
            pltpu.VMEM((_NT, 1), jnp.float32),
            pltpu.VMEM((_NT, 1), jnp.int32),
            pltpu.VMEM((_NT, 1), jnp.float32),
            pltpu.VMEM((_NT, 1), jnp.int32),
        ],
    )(flat, emb)


def _sc_gather(emb, ind):
    info = plsc.get_sparse_core_info()
    nc, ns = info.num_cores, info.num_subcores
    nw = nc * ns
    b = ind.shape[0]
    d = emb.shape[1]
    bpw = b // nw
    mesh = plsc.VectorSubcoreMesh(core_axis_name="c", subcore_axis_name="s")

    @functools.partial(
        pl.kernel,
        mesh=mesh,
        out_type=jax.ShapeDtypeStruct((b, d), jnp.float32),
        scratch_types=[
            pltpu.VMEM((bpw,), jnp.int32),
            pltpu.VMEM((bpw, d), jnp.float32),
            pltpu.SemaphoreType.DMA,
        ],
    )
    def gk(table_hbm, idx_hbm, out_hbm, idx_v, rows_v, sem):
        wid = lax.axis_index("s") * nc + lax.axis_index("c")
        base = wid * bpw
        pltpu.sync_copy(idx_hbm.at[pl.ds(base, bpw)], idx_v)
        pltpu.async_copy(table_hbm.at[idx_v], rows_v, sem).wait()
        pltpu.sync_copy(rows_v, out_hbm.at[pl.ds(base, bpw)])

    return gk(emb, ind)


def kernel(x, k, embed):
    b, n, d = x.shape
    kc = embed.shape[1]
    flat = x.reshape(b * n, d)
    emb = embed.reshape(kc, d)
    dist, i1, i2 = _dist_topk(flat, emb)
    ind = jnp.where(k == 0, i1[:, 0], i2[:, 0])
    quant = _sc_gather(emb, ind)
    quantize = quant.reshape(b, n, d)
    embed_ind = ind.reshape(b, n)
    dist_out = dist.reshape(1, b, n, kc)
    return quantize, embed_ind, dist_out


# speedup vs baseline: 52.7696x; 1.1125x over previous
import functools

import jax
import jax.numpy as jnp
from jax import lax
from jax.experimental import pallas as pl
from jax.experimental.pallas import tpu as pltpu
from jax.experimental.pallas import tpu_sc as plsc

_NT = 256
_KT = 8192
_BIG = 2 ** 30


def _dist_body(x_ref, e_ref, dist_ref, i1_ref, i2_ref, b1, bi1, b2, bi2):
    j = pl.program_id(1)
    nk = pl.num_programs(1)
    x = x_ref[...]
    e = e_ref[...]
    x2 = jnp.sum(x * x, axis=1, keepdims=True)
    y2 = jnp.sum(e * e, axis=1)[None, :]
    xy = lax.dot_general(x, e, (((1,), (1,)), ((), ())),
                         preferred_element_type=jnp.float32)
    sq = (x2 + y2) + (xy * -2.0)
    dist = -jnp.sqrt(jnp.maximum(sq, 0.0))
    dist_ref[...] = dist

    iota = lax.broadcasted_iota(jnp.int32, dist.shape, 1) + j * _KT
    v1 = jnp.max(dist, axis=1, keepdims=True)
    i1 = jnp.min(jnp.where(dist == v1, iota, _BIG), axis=1, keepdims=True)
    masked = jnp.where(iota == i1, -jnp.inf, dist)
    v2 = jnp.max(masked, axis=1, keepdims=True)
    i2 = jnp.min(jnp.where(masked == v2, iota, _BIG), axis=1, keepdims=True)

    @pl.when(j == 0)
    def _():
        b1[...] = v1
        bi1[...] = i1
        b2[...] = v2
        bi2[...] = i2

    @pl.when(j > 0)
    def _():
        rb1, ri1, rb2, ri2 = b1[...], bi1[...], b2[...], bi2[...]
        first_run = rb1 >= v1
        nb1 = jnp.where(first_run, rb1, v1)
        ni1 = jnp.where(first_run, ri1, i1)
        s_a = rb2 >= v1
        sva = jnp.where(s_a, rb2, v1)
        sia = jnp.where(s_a, ri2, i1)
        s_b = rb1 >= v2
        svb = jnp.where(s_b, rb1, v2)
        sib = jnp.where(s_b, ri1, i2)
        b1[...] = nb1
        bi1[...] = ni1
        b2[...] = jnp.where(first_run, sva, svb)
        bi2[...] = jnp.where(first_run, sia, sib)

    @pl.when(j == nk - 1)
    def _():
        i1_ref[...] = bi1[...]
        i2_ref[...] = bi2[...]


def _dist_topk(flat, emb):
    n, d = flat.shape
    kc = emb.shape[0]
    return pl.pallas_call(
        _dist_body,
        grid=(n // _NT, kc // _KT),
        in_specs=[
            pl.BlockSpec((_NT, d), lambda i, j: (i, 0)),
            pl.BlockSpec((_KT, d), lambda i, j: (j, 0)),
        ],
        out_specs=[
            pl.BlockSpec((_NT, _KT), lambda i, j: (i, j)),
            pl.BlockSpec((_NT, 1), lambda i, j: (i, 0)),
            pl.BlockSpec((_NT, 1), lambda i, j: (i, 0)),
        ],
        out_shape=[
            jax.ShapeDtypeStruct((n, kc), jnp.float32),
            jax.ShapeDtypeStruct((n, 1), jnp.int32),
            jax.ShapeDtypeStruct((n, 1), jnp.int32),
        ],
        scratch_shapes=[
            pltpu.VMEM((_NT, 1), jnp.float32),
            pltpu.VMEM((_NT, 1), jnp.int32),
            pltpu.VMEM((_NT, 1), jnp.float32),
            pltpu.VMEM((_NT, 1), jnp.int32),
        ],
    )(flat, emb)


def _sc_gather(emb, ind):
    info = plsc.get_sparse_core_info()
    nc, ns = info.num_cores, info.num_subcores
    nw = nc * ns
    b = ind.shape[0]
    d = emb.shape[1]
    bpw = b // nw
    mesh = plsc.VectorSubcoreMesh(core_axis_name="c", subcore_axis_name="s")

    @functools.partial(
        pl.kernel,
        mesh=mesh,
        out_type=jax.ShapeDtypeStruct((b, d), jnp.float32),
        scratch_types=[
            pltpu.VMEM((bpw,), jnp.int32),
            pltpu.VMEM((bpw, d), jnp.float32),
            pltpu.SemaphoreType.DMA,
        ],
    )
    def gk(table_hbm, idx_hbm, out_hbm, idx_v, rows_v, sem):
        wid = lax.axis_index("s") * nc + lax.axis_index("c")
        base = wid * bpw
        pltpu.sync_copy(idx_hbm.at[pl.ds(base, bpw)], idx_v)
        pltpu.async_copy(table_hbm.at[idx_v], rows_v, sem).wait()
        pltpu.sync_copy(rows_v, out_hbm.at[pl.ds(base, bpw)])

    return gk(emb, ind)


def kernel(x, k, embed):
    b, n, d = x.shape
    kc = embed.shape[1]
    flat = x.reshape(b * n, d)
    emb = embed.reshape(kc, d)
    dist, i1, i2 = _dist_topk(flat, emb)
    ind = jnp.where(k == 0, i1[:, 0], i2[:, 0])
    quant = _sc_gather(emb, ind)
    quantize = quant.reshape(b, n, d)
    embed_ind = ind.reshape(b, n)
    dist_out = dist.reshape(1, b, n, kc)
    return quantize, embed_ind, dist_out


# speedup vs baseline: 54.9798x; 1.0419x over previous
import functools

import jax
import jax.numpy as jnp
from jax import lax
from jax.experimental import pallas as pl
from jax.experimental.pallas import tpu as pltpu
from jax.experimental.pallas import tpu_sc as plsc

_NT = 256
_KT = 8192
_BIG = 2 ** 30


def _dist_body(x_ref, e_ref, x2_ref, y2_ref, dist_ref, i1_ref, i2_ref,
               b1, bi1, b2, bi2):
    j = pl.program_id(1)
    nk = pl.num_programs(1)
    x = x_ref[...]
    e = e_ref[...]
    x2 = x2_ref[...]
    y2 = y2_ref[...]
    xy = lax.dot_general(x, e, (((1,), (1,)), ((), ())),
                         preferred_element_type=jnp.float32)
    sq = (x2 + y2) + (xy * -2.0)
    dist = -jnp.sqrt(jnp.maximum(sq, 0.0))
    dist_ref[...] = dist

    iota = lax.broadcasted_iota(jnp.int32, dist.shape, 1) + j * _KT
    v1 = jnp.max(dist, axis=1, keepdims=True)
    i1 = jnp.min(jnp.where(dist == v1, iota, _BIG), axis=1, keepdims=True)
    masked = jnp.where(iota == i1, -jnp.inf, dist)
    v2 = jnp.max(masked, axis=1, keepdims=True)
    i2 = jnp.min(jnp.where(masked == v2, iota, _BIG), axis=1, keepdims=True)

    @pl.when(j == 0)
    def _():
        b1[...] = v1
        bi1[...] = i1
        b2[...] = v2
        bi2[...] = i2

    @pl.when(j > 0)
    def _():
        rb1, ri1, rb2, ri2 = b1[...], bi1[...], b2[...], bi2[...]
        first_run = rb1 >= v1
        nb1 = jnp.where(first_run, rb1, v1)
        ni1 = jnp.where(first_run, ri1, i1)
        s_a = rb2 >= v1
        sva = jnp.where(s_a, rb2, v1)
        sia = jnp.where(s_a, ri2, i1)
        s_b = rb1 >= v2
        svb = jnp.where(s_b, rb1, v2)
        sib = jnp.where(s_b, ri1, i2)
        b1[...] = nb1
        bi1[...] = ni1
        b2[...] = jnp.where(first_run, sva, svb)
        bi2[...] = jnp.where(first_run, sia, sib)

    @pl.when(j == nk - 1)
    def _():
        i1_ref[...] = bi1[...]
        i2_ref[...] = bi2[...]


def _dist_topk(flat, emb, x2, y2):
    n, d = flat.shape
    kc = emb.shape[0]
    return pl.pallas_call(
        _dist_body,
        grid=(n // _NT, kc // _KT),
        in_specs=[
            pl.BlockSpec((_NT, d), lambda i, j: (i, 0)),
            pl.BlockSpec((_KT, d), lambda i, j: (j, 0)),
            pl.BlockSpec((_NT, 1), lambda i, j: (i, 0)),
            pl.BlockSpec((1, _KT), lambda i, j: (0, j)),
        ],
        out_specs=[
            pl.BlockSpec((_NT, _KT), lambda i, j: (i, j)),
            pl.BlockSpec((_NT, 1), lambda i, j: (i, 0)),
            pl.BlockSpec((_NT, 1), lambda i, j: (i, 0)),
        ],
        out_shape=[
            jax.ShapeDtypeStruct((n, kc), jnp.float32),
            jax.ShapeDtypeStruct((n, 1), jnp.int32),
            jax.ShapeDtypeStruct((n, 1), jnp.int32),
        ],
        scratch_shapes=[
            pltpu.VMEM((_NT, 1), jnp.float32),
            pltpu.VMEM((_NT, 1), jnp.int32),
            pltpu.VMEM((_NT, 1), jnp.float32),
            pltpu.VMEM((_NT, 1), jnp.int32),
        ],
    )(flat, emb, x2, y2)


def _sc_gather(emb, ind):
    info = plsc.get_sparse_core_info()
    nc, ns = info.num_cores, info.num_subcores
    nw = nc * ns
    b = ind.shape[0]
    d = emb.shape[1]
    bpw = b // nw
    mesh = plsc.VectorSubcoreMesh(core_axis_name="c", subcore_axis_name="s")

    @functools.partial(
        pl.kernel,
        mesh=mesh,
        out_type=jax.ShapeDtypeStruct((b, d), jnp.float32),
        scratch_types=[
            pltpu.VMEM((bpw,), jnp.int32),
            pltpu.VMEM((bpw, d), jnp.float32),
            pltpu.SemaphoreType.DMA,
        ],
    )
    def gk(table_hbm, idx_hbm, out_hbm, idx_v, rows_v, sem):
        wid = lax.axis_index("s") * nc + lax.axis_index("c")
        base = wid * bpw
        pltpu.sync_copy(idx_hbm.at[pl.ds(base, bpw)], idx_v)
        pltpu.async_copy(table_hbm.at[idx_v], rows_v, sem).wait()
        pltpu.sync_copy(rows_v, out_hbm.at[pl.ds(base, bpw)])

    return gk(emb, ind)


def kernel(x, k, embed):
    b, n, d = x.shape
    kc = embed.shape[1]
    flat = x.reshape(b * n, d)
    emb = embed.reshape(kc, d)
    x2 = jnp.sum(flat ** 2, axis=-1)[:, None]
    y2 = jnp.sum(emb ** 2, axis=-1)[None, :]
    dist, i1, i2 = _dist_topk(flat, emb, x2, y2)
    ind = jnp.where(k == 0, i1[:, 0], i2[:, 0])
    quant = _sc_gather(emb, ind)
    quantize = quant.reshape(b, n, d)
    embed_ind = ind.reshape(b, n)
    dist_out = dist.reshape(1, b, n, kc)
    return quantize, embed_ind, dist_out
